# baseline (speedup 1.0000x reference)
import functools

import jax
import jax.numpy as jnp
from jax import lax
from jax.experimental import pallas as pl
from jax.experimental.pallas import tpu as pltpu
from jax.experimental.pallas import tpu_sc as plsc

B = 16384
D = 128
V = 100000

NC = 2
NS = 16
NW = NC * NS
LANES = 16

BPW = B // NW
CH = 128
NCHUNK = BPW // CH
KSTEP = D // LANES
MUNROLL = 2


def _sc_body(feat_hbm, lab_hbm, cent_hbm, out_hbm,
             idx_v, rows0, rows1, rows2, rows3, feat0, feat1, acc_v,
             gsem0, gsem1, gsem2, gsem3, fsem0, fsem1):
    wid = lax.axis_index("s") * NC + lax.axis_index("c")
    base = wid * BPW

    pltpu.sync_copy(lab_hbm.at[pl.ds(base, BPW)], idx_v)

    rows_bufs = (rows0, rows1, rows2, rows3)
    gsems = (gsem0, gsem1, gsem2, gsem3)
    feat_bufs = (feat0, feat1)
    fsems = (fsem0, fsem1)

    g_cp = [
        pltpu.async_copy(
            cent_hbm.at[idx_v.at[pl.ds(c * CH, CH)]], rows_bufs[c], gsems[c])
        for c in range(NCHUNK)
    ]

    def issue_feat(c):
        p = c % 2
        return pltpu.async_copy(
            feat_hbm.at[pl.ds(base + c * CH, CH)], feat_bufs[p], fsems[p])

    f_cp = [issue_feat(0), issue_feat(1)]

    zeros = jnp.zeros((LANES,), jnp.float32)
    m_acc = (zeros, zeros, zeros, zeros)
    for c in range(NCHUNK):
        g_cp[c].wait()
        f_cp[c].wait()
        rbuf = rows_bufs[c]
        fbuf = feat_bufs[c % 2]

        def mbody(i, carry):
            accs = list(carry)
            r0 = i * MUNROLL
            for u in range(MUNROLL):
                for k in range(KSTEP):
                    fv = fbuf[r0 + u, pl.ds(k * LANES, LANES)]
                    cv = rbuf[r0 + u, pl.ds(k * LANES, LANES)]
                    dd = fv - cv
                    accs[(k + 4 * u) % 4] = accs[(k + 4 * u) % 4] + dd * dd
            return tuple(accs)

        m_acc = lax.fori_loop(0, CH // MUNROLL, mbody, m_acc)
        if c + 2 < NCHUNK:
            f_cp.append(issue_feat(c + 2))

    m_vec = (m_acc[0] + m_acc[1]) + (m_acc[2] + m_acc[3])
    acc_v[...] = m_vec
    pltpu.sync_copy(acc_v, out_hbm.at[wid])


_sc_mse = functools.partial(
    pl.kernel,
    mesh=plsc.VectorSubcoreMesh(core_axis_name="c", subcore_axis_name="s"),
    out_type=jax.ShapeDtypeStruct((NW, LANES), jnp.float32),
    scratch_types=[
        pltpu.VMEM((BPW,), jnp.int32),
        pltpu.VMEM((CH, D), jnp.float32),
        pltpu.VMEM((CH, D), jnp.float32),
        pltpu.VMEM((CH, D), jnp.float32),
        pltpu.VMEM((CH, D), jnp.float32),
        pltpu.VMEM((CH, D), jnp.float32),
        pltpu.VMEM((CH, D), jnp.float32),
        pltpu.VMEM((LANES,), jnp.float32),
        pltpu.SemaphoreType.DMA,
        pltpu.SemaphoreType.DMA,
        pltpu.SemaphoreType.DMA,
        pltpu.SemaphoreType.DMA,
        pltpu.SemaphoreType.DMA,
        pltpu.SemaphoreType.DMA,
    ],
)(_sc_body)


RB = 20000
GRID = V // RB


def _tc_body(cent_ref, part_ref, loss_ref, var_ref, acc_ref):
    i = pl.program_id(0)

    @pl.when(i == 0)
    def _():
        acc_ref[...] = jnp.zeros_like(acc_ref)

    x = cent_ref[...]
    acc_ref[0:1, :] += jnp.sum(x, axis=0, keepdims=True)
    acc_ref[1:2, :] += jnp.sum(x * x, axis=0, keepdims=True)

    @pl.when(i == GRID - 1)
    def _():
        s = jnp.sum(acc_ref[0:1, :])
        ss = jnp.sum(acc_ref[1:2, :])
        n = jnp.float32(V * D)
        var_ref[0, 0] = (ss - s * (s / n)) / (n - 1.0)
        loss_ref[0, 0] = jnp.sum(part_ref[...]) / jnp.float32(B * D)


def _tc_finish(centers, partials):
    return pl.pallas_call(
        _tc_body,
        grid=(GRID,),
        in_specs=[
            pl.BlockSpec((RB, D), lambda i: (i, 0)),
            pl.BlockSpec((NW, LANES), lambda i: (0, 0)),
        ],
        out_specs=[
            pl.BlockSpec(memory_space=pltpu.SMEM),
            pl.BlockSpec(memory_space=pltpu.SMEM),
        ],
        out_shape=[
            jax.ShapeDtypeStruct((1, 1), jnp.float32),
            jax.ShapeDtypeStruct((1, 1), jnp.float32),
        ],
        scratch_shapes=[pltpu.VMEM((2, D), jnp.float32)],
    )(centers, partials)


def kernel(features, labels, centers):
    labels32 = labels.astype(jnp.int32)
    partials = _sc_mse(features, labels32, centers)
    loss, var = _tc_finish(centers, partials)
    return (loss[0, 0], var[0, 0])


# speedup vs baseline: 1.0135x; 1.0135x over previous
import functools

import jax
import jax.numpy as jnp
from jax import lax
from jax.experimental import pallas as pl
from jax.experimental.pallas import tpu as pltpu
from jax.experimental.pallas import tpu_sc as plsc

B = 16384
D = 128
V = 100000

NC = 2
NS = 16
NW = NC * NS
LANES = 16

BPW = B // NW
CH = 128
NCHUNK = BPW // CH
KSTEP = D // LANES
MUNROLL = 2


def _sc_body(feat_hbm, lab_hbm, cent_hbm, out_hbm,
             idx_v, rows0, rows1, rows2, rows3, feat0, feat1, acc_v,
             gsem0, gsem1, gsem2, gsem3, fsem0, fsem1):
    wid = lax.axis_index("s") * NC + lax.axis_index("c")
    base = wid * BPW

    pltpu.sync_copy(lab_hbm.at[pl.ds(base, BPW)], idx_v)

    rows_bufs = (rows0, rows1, rows2, rows3)
    gsems = (gsem0, gsem1, gsem2, gsem3)
    feat_bufs = (feat0, feat1)
    fsems = (fsem0, fsem1)

    def issue_gather(c):
        p = c % 2
        return pltpu.async_copy(
            cent_hbm.at[idx_v.at[pl.ds(c * CH, CH)]], rows_bufs[p], gsems[p])

    def issue_feat(c):
        p = c % 2
        return pltpu.async_copy(
            feat_hbm.at[pl.ds(base + c * CH, CH)], feat_bufs[p], fsems[p])

    g_cp = [issue_gather(0), issue_gather(1)]
    f_cp = [issue_feat(0), issue_feat(1)]

    zeros = jnp.zeros((LANES,), jnp.float32)
    m_acc = (zeros, zeros, zeros, zeros)
    for c in range(NCHUNK):
        g_cp[c].wait()
        f_cp[c].wait()
        rbuf = rows_bufs[c % 2]
        fbuf = feat_bufs[c % 2]

        def mbody(i, carry):
            accs = list(carry)
            r0 = i * MUNROLL
            for u in range(MUNROLL):
                for k in range(KSTEP):
                    fv = fbuf[r0 + u, pl.ds(k * LANES, LANES)]
                    cv = rbuf[r0 + u, pl.ds(k * LANES, LANES)]
                    dd = fv - cv
                    accs[(k + 4 * u) % 4] = accs[(k + 4 * u) % 4] + dd * dd
            return tuple(accs)

        m_acc = lax.fori_loop(0, CH // MUNROLL, mbody, m_acc)
        if c + 2 < NCHUNK:
            g_cp.append(issue_gather(c + 2))
            f_cp.append(issue_feat(c + 2))

    m_vec = (m_acc[0] + m_acc[1]) + (m_acc[2] + m_acc[3])
    acc_v[...] = m_vec
    pltpu.sync_copy(acc_v, out_hbm.at[wid])


_sc_mse = functools.partial(
    pl.kernel,
    mesh=plsc.VectorSubcoreMesh(core_axis_name="c", subcore_axis_name="s"),
    out_type=jax.ShapeDtypeStruct((NW, LANES), jnp.float32),
    scratch_types=[
        pltpu.VMEM((BPW,), jnp.int32),
        pltpu.VMEM((CH, D), jnp.float32),
        pltpu.VMEM((CH, D), jnp.float32),
        pltpu.VMEM((CH, D), jnp.float32),
        pltpu.VMEM((CH, D), jnp.float32),
        pltpu.VMEM((CH, D), jnp.float32),
        pltpu.VMEM((CH, D), jnp.float32),
        pltpu.VMEM((LANES,), jnp.float32),
        pltpu.SemaphoreType.DMA,
        pltpu.SemaphoreType.DMA,
        pltpu.SemaphoreType.DMA,
        pltpu.SemaphoreType.DMA,
        pltpu.SemaphoreType.DMA,
        pltpu.SemaphoreType.DMA,
    ],
)(_sc_body)


RB = 20000
GRID = V // RB


def _tc_body(cent_ref, part_ref, loss_ref, var_ref, acc_ref):
    i = pl.program_id(0)

    @pl.when(i == 0)
    def _():
        acc_ref[...] = jnp.zeros_like(acc_ref)

    x = cent_ref[...]
    acc_ref[0:1, :] += jnp.sum(x, axis=0, keepdims=True)
    acc_ref[1:2, :] += jnp.sum(x * x, axis=0, keepdims=True)

    @pl.when(i == GRID - 1)
    def _():
        s = jnp.sum(acc_ref[0:1, :])
        ss = jnp.sum(acc_ref[1:2, :])
        n = jnp.float32(V * D)
        var_ref[0, 0] = (ss - s * (s / n)) / (n - 1.0)
        loss_ref[0, 0] = jnp.sum(part_ref[...]) / jnp.float32(B * D)


def _tc_finish(centers, partials):
    return pl.pallas_call(
        _tc_body,
        grid=(GRID,),
        in_specs=[
            pl.BlockSpec((RB, D), lambda i: (i, 0)),
            pl.BlockSpec((NW, LANES), lambda i: (0, 0)),
        ],
        out_specs=[
            pl.BlockSpec(memory_space=pltpu.SMEM),
            pl.BlockSpec(memory_space=pltpu.SMEM),
        ],
        out_shape=[
            jax.ShapeDtypeStruct((1, 1), jnp.float32),
            jax.ShapeDtypeStruct((1, 1), jnp.float32),
        ],
        scratch_shapes=[pltpu.VMEM((2, D), jnp.float32)],
    )(centers, partials)


def kernel(features, labels, centers):
    labels32 = labels.astype(jnp.int32)
    partials = _sc_mse(features, labels32, centers)
    loss, var = _tc_finish(centers, partials)
    return (loss[0, 0], var[0, 0])


# speedup vs baseline: 1.1055x; 1.0908x over previous
import functools

import jax
import jax.numpy as jnp
from jax import lax
from jax.experimental import pallas as pl
from jax.experimental.pallas import tpu as pltpu
from jax.experimental.pallas import tpu_sc as plsc

B = 16384
D = 128
V = 100000

NC = 2
NS = 16
NW = NC * NS
LANES = 16

BPW = B // NW
CH = 64
NCHUNK = BPW // CH
NBUF = 4
KSTEP = D // LANES
MUNROLL = 2


def _sc_body(feat_hbm, lab_hbm, cent_hbm, out_hbm,
             idx_v, rows0, rows1, rows2, rows3, feat0, feat1, feat2, feat3,
             acc_v, gsem0, gsem1, gsem2, gsem3, fsem0, fsem1, fsem2, fsem3):
    wid = lax.axis_index("s") * NC + lax.axis_index("c")
    base = wid * BPW

    pltpu.sync_copy(lab_hbm.at[pl.ds(base, BPW)], idx_v)

    rows_bufs = (rows0, rows1, rows2, rows3)
    gsems = (gsem0, gsem1, gsem2, gsem3)
    feat_bufs = (feat0, feat1, feat2, feat3)
    fsems = (fsem0, fsem1, fsem2, fsem3)

    def issue(c):
        p = c % NBUF
        g = pltpu.async_copy(
            cent_hbm.at[idx_v.at[pl.ds(c * CH, CH)]], rows_bufs[p], gsems[p])
        f = pltpu.async_copy(
            feat_hbm.at[pl.ds(base + c * CH, CH)], feat_bufs[p], fsems[p])
        return g, f

    cps = [issue(c) for c in range(NBUF)]

    zeros = jnp.zeros((LANES,), jnp.float32)
    m_acc = (zeros, zeros, zeros, zeros)
    for c in range(NCHUNK):
        p = c % NBUF
        g, f = cps[c]
        g.wait()
        f.wait()
        rbuf = rows_bufs[p]
        fbuf = feat_bufs[p]

        def mbody(i, carry):
            accs = list(carry)
            r0 = i * MUNROLL
            for u in range(MUNROLL):
                for k in range(KSTEP):
                    fv = fbuf[r0 + u, pl.ds(k * LANES, LANES)]
                    cv = rbuf[r0 + u, pl.ds(k * LANES, LANES)]
                    dd = fv - cv
                    accs[(k + 4 * u) % 4] = accs[(k + 4 * u) % 4] + dd * dd
            return tuple(accs)

        m_acc = lax.fori_loop(0, CH // MUNROLL, mbody, m_acc)
        if c + NBUF < NCHUNK:
            cps.append(issue(c + NBUF))

    m_vec = (m_acc[0] + m_acc[1]) + (m_acc[2] + m_acc[3])
    acc_v[...] = m_vec
    pltpu.sync_copy(acc_v, out_hbm.at[wid])


_sc_mse = functools.partial(
    pl.kernel,
    mesh=plsc.VectorSubcoreMesh(core_axis_name="c", subcore_axis_name="s"),
    out_type=jax.ShapeDtypeStruct((NW, LANES), jnp.float32),
    scratch_types=[
        pltpu.VMEM((BPW,), jnp.int32),
        pltpu.VMEM((CH, D), jnp.float32),
        pltpu.VMEM((CH, D), jnp.float32),
        pltpu.VMEM((CH, D), jnp.float32),
        pltpu.VMEM((CH, D), jnp.float32),
        pltpu.VMEM((CH, D), jnp.float32),
        pltpu.VMEM((CH, D), jnp.float32),
        pltpu.VMEM((CH, D), jnp.float32),
        pltpu.VMEM((CH, D), jnp.float32),
        pltpu.VMEM((LANES,), jnp.float32),
        pltpu.SemaphoreType.DMA,
        pltpu.SemaphoreType.DMA,
        pltpu.SemaphoreType.DMA,
        pltpu.SemaphoreType.DMA,
        pltpu.SemaphoreType.DMA,
        pltpu.SemaphoreType.DMA,
        pltpu.SemaphoreType.DMA,
        pltpu.SemaphoreType.DMA,
    ],
)(_sc_body)


RB = 25000
GRID = V // RB


def _tc_var_body(cent_ref, s_ref, ss_ref, acc_ref):
    i = pl.program_id(0)

    @pl.when(i == 0)
    def _():
        acc_ref[...] = jnp.zeros_like(acc_ref)

    x = cent_ref[...]
    acc_ref[0:1, :] += jnp.sum(x, axis=0, keepdims=True)
    acc_ref[1:2, :] += jnp.sum(x * x, axis=0, keepdims=True)

    @pl.when(i == GRID - 1)
    def _():
        s_ref[0, 0] = jnp.sum(acc_ref[0:1, :])
        ss_ref[0, 0] = jnp.sum(acc_ref[1:2, :])


def _tc_var(centers):
    return pl.pallas_call(
        _tc_var_body,
        grid=(GRID,),
        in_specs=[pl.BlockSpec((RB, D), lambda i: (i, 0))],
        out_specs=[
            pl.BlockSpec(memory_space=pltpu.SMEM),
            pl.BlockSpec(memory_space=pltpu.SMEM),
        ],
        out_shape=[
            jax.ShapeDtypeStruct((1, 1), jnp.float32),
            jax.ShapeDtypeStruct((1, 1), jnp.float32),
        ],
        scratch_shapes=[pltpu.VMEM((2, D), jnp.float32)],
    )(centers)


def kernel(features, labels, centers):
    labels32 = labels.astype(jnp.int32)
    partials = _sc_mse(features, labels32, centers)
    s, ss = _tc_var(centers)
    loss = jnp.sum(partials) / (B * D)
    n = V * D
    total = s[0, 0]
    mean = total / n
    var = (ss[0, 0] - total * mean) / (n - 1)
    return (loss, var)
